# Initial kernel scaffold; baseline (speedup 1.0000x reference)
import jax
import jax.numpy as jnp
from jax.experimental import pallas as pl


def kernel(x, Wg, bg, W1, b1, W2, b2):
    raise NotImplementedError("write your pallas kernel here")


# speedup vs baseline: 1.6515x; 1.6515x over previous
import jax
import jax.numpy as jnp
from jax.experimental import pallas as pl
from jax.experimental.pallas import tpu as pltpu

E = 64
K = 2
D = 768
F = 1536
T = 128


def _moe_body(x_ref, Wg_ref, bg_ref, W1_ref, b1_ref, W2_ref, b2_ref,
              out_ref, w_ref, acc_ref):
    e = pl.program_id(0)

    @pl.when(e == 0)
    def _gating():
        xb = x_ref[:]
        logits = (jnp.dot(xb, Wg_ref[:], preferred_element_type=jnp.float32)
                  + bg_ref[0, :])
        probs = jax.nn.softmax(logits, axis=-1)
        eidx = jax.lax.broadcasted_iota(jnp.int32, (T, E), 1)
        m1 = jnp.max(probs, axis=1, keepdims=True)
        i1 = jnp.argmax(probs, axis=1)[:, None]
        probs2 = jnp.where(eidx == i1, -jnp.inf, probs)
        m2 = jnp.max(probs2, axis=1, keepdims=True)
        i2 = jnp.argmax(probs2, axis=1)[:, None]
        denom = m1 + m2
        w = (jnp.where(eidx == i1, m1, 0.0)
             + jnp.where(eidx == i2, m2, 0.0)) / denom
        w_ref[:] = w
        acc_ref[:] = jnp.zeros_like(acc_ref)

    xb = x_ref[:]
    h = jnp.maximum(
        jnp.dot(xb, W1_ref[0], preferred_element_type=jnp.float32)
        + b1_ref[0, 0, :], 0.0)
    o = jnp.dot(h, W2_ref[0], preferred_element_type=jnp.float32)
    eidx = jax.lax.broadcasted_iota(jnp.int32, (T, E), 1)
    wcol = jnp.sum(jnp.where(eidx == e, w_ref[:], 0.0), axis=1, keepdims=True)
    acc_ref[:] += wcol * o

    @pl.when(e == E - 1)
    def _finish():
        out_ref[:] = acc_ref[:] + jnp.dot(
            w_ref[:], b2_ref[:], preferred_element_type=jnp.float32)


def kernel(x, Wg, bg, W1, b1, W2, b2):
    B, S, _ = x.shape
    xf = x.reshape(T, D)
    bg2 = bg.reshape(1, E)
    b1r = b1.reshape(E, 1, F)
    out = pl.pallas_call(
        _moe_body,
        grid=(E,),
        in_specs=[
            pl.BlockSpec((T, D), lambda e: (0, 0)),
            pl.BlockSpec((D, E), lambda e: (0, 0)),
            pl.BlockSpec((1, E), lambda e: (0, 0)),
            pl.BlockSpec((1, D, F), lambda e: (e, 0, 0)),
            pl.BlockSpec((1, 1, F), lambda e: (e, 0, 0)),
            pl.BlockSpec((1, F, D), lambda e: (e, 0, 0)),
            pl.BlockSpec((E, D), lambda e: (0, 0)),
        ],
        out_specs=pl.BlockSpec((T, D), lambda e: (0, 0)),
        out_shape=jax.ShapeDtypeStruct((T, D), jnp.float32),
        scratch_shapes=[
            pltpu.VMEM((T, E), jnp.float32),
            pltpu.VMEM((T, D), jnp.float32),
        ],
    )(xf, Wg, bg2, W1, b1r, W2, b2)
    return out.reshape(B, S, D)
